# Initial kernel scaffold; baseline (speedup 1.0000x reference)
import jax
import jax.numpy as jnp
from jax.experimental import pallas as pl


def kernel(token_ids, weight):
    raise NotImplementedError("write your pallas kernel here")


# speedup vs baseline: 1.5116x; 1.5116x over previous
import functools

import jax
import jax.numpy as jnp
from jax import lax
from jax.experimental import pallas as pl
from jax.experimental.pallas import tpu as pltpu
from jax.experimental.pallas import tpu_sc as plsc

NC = 2
NS = 16
NW = NC * NS
D = 32
CHUNK = 1024


@functools.partial(jax.jit, static_argnames=("b_total",))
def _embed_lookup(idx_flat, weight, b_total):
    b_per_w = b_total // NW
    n_chunks = b_per_w // CHUNK
    mesh = plsc.VectorSubcoreMesh(core_axis_name="c", subcore_axis_name="s")

    @functools.partial(
        pl.kernel,
        out_type=jax.ShapeDtypeStruct((b_total, D), jnp.float32),
        mesh=mesh,
        scratch_types=[
            pltpu.VMEM((b_per_w,), jnp.int32),
            pltpu.VMEM((2, CHUNK, D), jnp.float32),
            pltpu.SemaphoreType.DMA,
            pltpu.SemaphoreType.DMA,
        ],
        compiler_params=pltpu.CompilerParams(use_tc_tiling_on_sc=False),
    )
    def k(idx_hbm, table_hbm, out_hbm, idx_v, rows_v, gsem0, gsem1):
        wid = lax.axis_index("s") * NC + lax.axis_index("c")
        base = wid * b_per_w
        pltpu.sync_copy(idx_hbm.at[pl.ds(base, b_per_w)], idx_v)
        gsems = (gsem0, gsem1)
        g = [None] * n_chunks
        g[0] = pltpu.async_copy(
            table_hbm.at[idx_v.at[pl.ds(0, CHUNK)]], rows_v.at[0], gsems[0])
        for c in range(n_chunks):
            nxt = c + 1
            if nxt < n_chunks:
                g[nxt] = pltpu.async_copy(
                    table_hbm.at[idx_v.at[pl.ds(nxt * CHUNK, CHUNK)]],
                    rows_v.at[nxt % 2], gsems[nxt % 2])
            g[c].wait()
            pltpu.sync_copy(rows_v.at[c % 2],
                            out_hbm.at[pl.ds(base + c * CHUNK, CHUNK)])

    return k(idx_flat, weight)


def kernel(token_ids, weight):
    b, s = token_ids.shape
    idx_flat = token_ids.reshape(-1).astype(jnp.int32)
    out = _embed_lookup(idx_flat, weight, b * s)
    return out.reshape(b, s, D)
